# initial kernel scaffold (unmeasured)
import jax
import jax.numpy as jnp
from jax import lax
from jax.experimental import pallas as pl
from jax.experimental.pallas import tpu as pltpu

N_DEV = 4


def kernel(x, w_mat, scale_x, scale_w):
    m_total, k_shard = x.shape
    k_total, n = w_mat.shape
    m_per = m_total // N_DEV
    assert k_total == N_DEV * k_shard

    def body(x_ref, w_ref, sx_ref, sw_ref, out_ref,
             send_buf, recv_buf, send_sems, recv_sems, copy_sem):
        my_i = lax.axis_index("i")

        for p in range(N_DEV):
            send_buf[p] = x_ref[p * m_per:(p + 1) * m_per, :].astype(
                jnp.float8_e5m2)

        local = pltpu.make_async_copy(
            send_buf.at[my_i], recv_buf.at[my_i], copy_sem)
        local.start()

        barrier_sem = pltpu.get_barrier_semaphore()
        for h in range(1, N_DEV):
            peer = lax.rem(my_i + h, N_DEV)
            pl.semaphore_signal(
                barrier_sem, inc=1,
                device_id=(peer,), device_id_type=pl.DeviceIdType.MESH)
        pl.semaphore_wait(barrier_sem, N_DEV - 1)

        rdmas = []
        for h in range(1, N_DEV):
            peer = lax.rem(my_i + h, N_DEV)
            rdma = pltpu.make_async_remote_copy(
                src_ref=send_buf.at[peer],
                dst_ref=recv_buf.at[my_i],
                send_sem=send_sems.at[h - 1],
                recv_sem=recv_sems.at[h - 1],
                device_id=(peer,),
                device_id_type=pl.DeviceIdType.MESH,
            )
            rdma.start()
            rdmas.append(rdma)

        local.wait()
        for rdma in rdmas:
            rdma.wait()

        scale = sx_ref[0] * sw_ref[0]
        acc = jnp.zeros((m_per, n), jnp.float32)
        for j in range(N_DEV):
            a = recv_buf[j].astype(jnp.bfloat16)
            b = w_ref[j * k_shard:(j + 1) * k_shard, :].astype(jnp.bfloat16)
            acc = acc + jnp.dot(a, b, preferred_element_type=jnp.float32)
        out_ref[:, :] = acc * scale

    return pl.pallas_call(
        body,
        out_shape=jax.ShapeDtypeStruct((m_per, n), jnp.float32),
        in_specs=[
            pl.BlockSpec(memory_space=pltpu.VMEM),
            pl.BlockSpec(memory_space=pltpu.VMEM),
            pl.BlockSpec(memory_space=pltpu.SMEM),
            pl.BlockSpec(memory_space=pltpu.SMEM),
        ],
        out_specs=pl.BlockSpec(memory_space=pltpu.VMEM),
        scratch_shapes=[
            pltpu.VMEM((N_DEV, m_per, k_shard), jnp.float8_e5m2),
            pltpu.VMEM((N_DEV, m_per, k_shard), jnp.float8_e5m2),
            pltpu.SemaphoreType.DMA((N_DEV - 1,)),
            pltpu.SemaphoreType.DMA((N_DEV - 1,)),
            pltpu.SemaphoreType.DMA,
        ],
        compiler_params=pltpu.CompilerParams(
            collective_id=0,
            vmem_limit_bytes=128 * 1024 * 1024,
        ),
    )(x, w_mat, scale_x, scale_w)


# baseline (device time: 66912 ns/iter reference)
import jax
import jax.numpy as jnp
from jax import lax
from jax.experimental import pallas as pl
from jax.experimental.pallas import tpu as pltpu

N_DEV = 4


def kernel(x, w_mat, scale_x, scale_w):
    m_total, k_shard = x.shape
    k_total, n = w_mat.shape
    m_per = m_total // N_DEV
    assert k_total == N_DEV * k_shard

    def body(x_hbm, w_hbm, sx_ref, sw_ref, out_ref,
             send_buf, recv_buf, xtmp, wtmp,
             send_sems, recv_sems, xsems, wsems, copy_sem):
        my_i = lax.axis_index("i")

        xdma = [
            pltpu.make_async_copy(
                x_hbm.at[pl.ds(p * m_per, m_per), :],
                xtmp.at[p % 2], xsems.at[p % 2])
            for p in range(N_DEV)
        ]
        wdma = [
            pltpu.make_async_copy(
                w_hbm.at[pl.ds(j * k_shard, k_shard), :],
                wtmp.at[j % 2], wsems.at[j % 2])
            for j in range(N_DEV)
        ]
        xdma[0].start()
        xdma[1].start()
        wdma[0].start()
        wdma[1].start()

        for p in range(N_DEV):
            xdma[p].wait()
            send_buf[p] = xtmp[p % 2].astype(jnp.float8_e5m2)
            if p + 2 < N_DEV:
                xdma[p + 2].start()

        local = pltpu.make_async_copy(
            send_buf.at[my_i], recv_buf.at[my_i], copy_sem)
        local.start()

        barrier_sem = pltpu.get_barrier_semaphore()
        for h in range(1, N_DEV):
            peer = lax.rem(my_i + h, N_DEV)
            pl.semaphore_signal(
                barrier_sem, inc=1,
                device_id=(peer,), device_id_type=pl.DeviceIdType.MESH)
        pl.semaphore_wait(barrier_sem, N_DEV - 1)

        rdmas = []
        for h in range(1, N_DEV):
            peer = lax.rem(my_i + h, N_DEV)
            rdma = pltpu.make_async_remote_copy(
                src_ref=send_buf.at[peer],
                dst_ref=recv_buf.at[my_i],
                send_sem=send_sems.at[h - 1],
                recv_sem=recv_sems.at[h - 1],
                device_id=(peer,),
                device_id_type=pl.DeviceIdType.MESH,
            )
            rdma.start()
            rdmas.append(rdma)

        local.wait()
        for rdma in rdmas:
            rdma.wait()

        for j in range(N_DEV):
            wdma[j].wait()
            a = recv_buf[j].astype(jnp.bfloat16)
            b = wtmp[j % 2].astype(jnp.bfloat16)
            part = jnp.dot(a, b, preferred_element_type=jnp.float32)
            if j == 0:
                out_ref[:, :] = part
            else:
                out_ref[:, :] = out_ref[:, :] + part
            if j + 2 < N_DEV:
                wdma[j + 2].start()

        scale = sx_ref[0] * sw_ref[0]
        out_ref[:, :] = out_ref[:, :] * scale

    return pl.pallas_call(
        body,
        out_shape=jax.ShapeDtypeStruct((m_per, n), jnp.float32),
        in_specs=[
            pl.BlockSpec(memory_space=pl.ANY),
            pl.BlockSpec(memory_space=pl.ANY),
            pl.BlockSpec(memory_space=pltpu.SMEM),
            pl.BlockSpec(memory_space=pltpu.SMEM),
        ],
        out_specs=pl.BlockSpec(memory_space=pltpu.VMEM),
        scratch_shapes=[
            pltpu.VMEM((N_DEV, m_per, k_shard), jnp.float8_e5m2),
            pltpu.VMEM((N_DEV, m_per, k_shard), jnp.float8_e5m2),
            pltpu.VMEM((2, m_per, k_shard), jnp.float32),
            pltpu.VMEM((2, k_shard, n), jnp.float32),
            pltpu.SemaphoreType.DMA((N_DEV - 1,)),
            pltpu.SemaphoreType.DMA((N_DEV - 1,)),
            pltpu.SemaphoreType.DMA((2,)),
            pltpu.SemaphoreType.DMA((2,)),
            pltpu.SemaphoreType.DMA,
        ],
        compiler_params=pltpu.CompilerParams(
            collective_id=0,
            vmem_limit_bytes=64 * 1024 * 1024,
        ),
    )(x, w_mat, scale_x, scale_w)


# device time: 48428 ns/iter; 1.3817x vs baseline; 1.3817x over previous
import jax
import jax.numpy as jnp
from jax import lax
from jax.experimental import pallas as pl
from jax.experimental.pallas import tpu as pltpu

N_DEV = 4


def kernel(x, w_mat, scale_x, scale_w):
    m_total, k_shard = x.shape
    k_total, n = w_mat.shape
    m_per = m_total // N_DEV
    assert k_total == N_DEV * k_shard

    def body(x_hbm, w_hbm, sx_ref, sw_ref, out_ref,
             send_buf, recv_buf, xtmp, wtmp,
             send_sems, recv_sems, xsems, wsems):
        my_i = lax.axis_index("i")
        scale = sx_ref[0] * sw_ref[0]

        barrier_sem = pltpu.get_barrier_semaphore()
        for h in range(1, N_DEV):
            peer = lax.rem(my_i + h, N_DEV)
            pl.semaphore_signal(
                barrier_sem, inc=1,
                device_id=(peer,), device_id_type=pl.DeviceIdType.MESH)

        def xblk(h, slot):
            peer = lax.rem(my_i + h, N_DEV)
            return pltpu.make_async_copy(
                x_hbm.at[pl.ds(peer * m_per, m_per), :],
                xtmp.at[slot], xsems.at[slot])

        def wblk(h, slot):
            src = lax.rem(my_i - h + N_DEV, N_DEV)
            return pltpu.make_async_copy(
                w_hbm.at[pl.ds(src * k_shard, k_shard), :],
                wtmp.at[slot], wsems.at[slot])

        xdma = [xblk(1, 0), xblk(2, 1), xblk(3, 0), xblk(0, 1)]
        wdma = [wblk(0, 0), wblk(1, 1), wblk(2, 0), wblk(3, 1)]

        xdma[0].start()
        xdma[1].start()
        wdma[0].start()

        pl.semaphore_wait(barrier_sem, N_DEV - 1)

        rdmas = []
        for h in range(1, N_DEV):
            peer = lax.rem(my_i + h, N_DEV)
            xdma[h - 1].wait()
            send_buf[h - 1] = xtmp[(h - 1) % 2].astype(jnp.float8_e5m2)
            rdma = pltpu.make_async_remote_copy(
                src_ref=send_buf.at[h - 1],
                dst_ref=recv_buf.at[h - 1],
                send_sem=send_sems.at[h - 1],
                recv_sem=recv_sems.at[h - 1],
                device_id=(peer,),
                device_id_type=pl.DeviceIdType.MESH,
            )
            rdma.start()
            rdmas.append(rdma)
            if h + 1 < N_DEV:
                xdma[h + 1].start()
        wdma[1].start()

        xdma[3].wait()
        wdma[0].wait()
        a = (xtmp[1] * scale).astype(jnp.bfloat16)
        b = wtmp[0].astype(jnp.bfloat16)
        wdma[2].start()
        out_ref[:, :] = jnp.dot(a, b, preferred_element_type=jnp.float32)

        for h in range(1, N_DEV):
            rdmas[h - 1].wait_recv()
            wdma[h].wait()
            a = (recv_buf[h - 1].astype(jnp.float32) * scale).astype(
                jnp.bfloat16)
            b = wtmp[h % 2].astype(jnp.bfloat16)
            if h == 1:
                wdma[3].start()
            out_ref[:, :] = out_ref[:, :] + jnp.dot(
                a, b, preferred_element_type=jnp.float32)

        for rdma in rdmas:
            rdma.wait_send()

    return pl.pallas_call(
        body,
        out_shape=jax.ShapeDtypeStruct((m_per, n), jnp.float32),
        in_specs=[
            pl.BlockSpec(memory_space=pl.ANY),
            pl.BlockSpec(memory_space=pl.ANY),
            pl.BlockSpec(memory_space=pltpu.SMEM),
            pl.BlockSpec(memory_space=pltpu.SMEM),
        ],
        out_specs=pl.BlockSpec(memory_space=pltpu.VMEM),
        scratch_shapes=[
            pltpu.VMEM((N_DEV - 1, m_per, k_shard), jnp.float8_e5m2),
            pltpu.VMEM((N_DEV - 1, m_per, k_shard), jnp.float8_e5m2),
            pltpu.VMEM((2, m_per, k_shard), jnp.float32),
            pltpu.VMEM((2, k_shard, n), jnp.float32),
            pltpu.SemaphoreType.DMA((N_DEV - 1,)),
            pltpu.SemaphoreType.DMA((N_DEV - 1,)),
            pltpu.SemaphoreType.DMA((2,)),
            pltpu.SemaphoreType.DMA((2,)),
        ],
        compiler_params=pltpu.CompilerParams(
            collective_id=0,
            vmem_limit_bytes=64 * 1024 * 1024,
        ),
    )(x, w_mat, scale_x, scale_w)


# device time: 43837 ns/iter; 1.5264x vs baseline; 1.1047x over previous
import jax
import jax.numpy as jnp
from jax import lax
from jax.experimental import pallas as pl
from jax.experimental.pallas import tpu as pltpu

N_DEV = 4


def kernel(x, w_mat, scale_x, scale_w):
    m_total, k_shard = x.shape
    k_total, n = w_mat.shape
    m_per = m_total // N_DEV
    assert k_total == N_DEV * k_shard

    def body(x_hbm, w_hbm, sx_ref, sw_ref, out_ref,
             send_buf, recv_buf, xtmp, wtmp,
             send_sems, recv_sems, xsems, wsems):
        my_i = lax.axis_index("i")
        scale = sx_ref[0] * sw_ref[0]

        barrier_sem = pltpu.get_barrier_semaphore()
        for h in range(1, N_DEV):
            peer = lax.rem(my_i + h, N_DEV)
            pl.semaphore_signal(
                barrier_sem, inc=1,
                device_id=(peer,), device_id_type=pl.DeviceIdType.MESH)

        def xblk(h, slot):
            peer = lax.rem(my_i + h, N_DEV)
            return pltpu.make_async_copy(
                x_hbm.at[pl.ds(peer * m_per, m_per), :],
                xtmp.at[slot], xsems.at[slot])

        def wblk(h, slot):
            src = lax.rem(my_i - h + N_DEV, N_DEV)
            return pltpu.make_async_copy(
                w_hbm.at[pl.ds(src * k_shard, k_shard), :],
                wtmp.at[slot], wsems.at[slot])

        xdma = [xblk(1, 0), xblk(2, 1), xblk(3, 0), xblk(0, 1)]
        wdma = [wblk(0, 0), wblk(1, 1), wblk(2, 0), wblk(3, 1)]

        xdma[0].start()
        xdma[1].start()
        wdma[0].start()

        pl.semaphore_wait(barrier_sem, N_DEV - 1)

        rdmas = []
        for h in range(1, N_DEV):
            peer = lax.rem(my_i + h, N_DEV)
            xdma[h - 1].wait()
            send_buf[h - 1] = xtmp[(h - 1) % 2].astype(jnp.float8_e5m2)
            rdma = pltpu.make_async_remote_copy(
                src_ref=send_buf.at[h - 1],
                dst_ref=recv_buf.at[h - 1],
                send_sem=send_sems.at[h - 1],
                recv_sem=recv_sems.at[h - 1],
                device_id=(peer,),
                device_id_type=pl.DeviceIdType.MESH,
            )
            rdma.start()
            rdmas.append(rdma)
            if h + 1 < N_DEV:
                xdma[h + 1].start()
        wdma[1].start()

        xdma[3].wait()
        wdma[0].wait()
        a = xtmp[1].astype(jnp.float8_e5m2)
        b = wtmp[0].astype(jnp.float8_e5m2)
        wdma[2].start()
        out_ref[:, :] = jnp.dot(a, b, preferred_element_type=jnp.float32)

        for h in range(1, N_DEV):
            rdmas[h - 1].wait_recv()
            wdma[h].wait()
            b = wtmp[h % 2].astype(jnp.float8_e5m2)
            if h == 1:
                wdma[3].start()
            out_ref[:, :] = out_ref[:, :] + jnp.dot(
                recv_buf[h - 1], b, preferred_element_type=jnp.float32)

        out_ref[:, :] = out_ref[:, :] * scale

        for rdma in rdmas:
            rdma.wait_send()

    return pl.pallas_call(
        body,
        out_shape=jax.ShapeDtypeStruct((m_per, n), jnp.float32),
        in_specs=[
            pl.BlockSpec(memory_space=pl.ANY),
            pl.BlockSpec(memory_space=pl.ANY),
            pl.BlockSpec(memory_space=pltpu.SMEM),
            pl.BlockSpec(memory_space=pltpu.SMEM),
        ],
        out_specs=pl.BlockSpec(memory_space=pltpu.VMEM),
        scratch_shapes=[
            pltpu.VMEM((N_DEV - 1, m_per, k_shard), jnp.float8_e5m2),
            pltpu.VMEM((N_DEV - 1, m_per, k_shard), jnp.float8_e5m2),
            pltpu.VMEM((2, m_per, k_shard), jnp.float32),
            pltpu.VMEM((2, k_shard, n), jnp.float32),
            pltpu.SemaphoreType.DMA((N_DEV - 1,)),
            pltpu.SemaphoreType.DMA((N_DEV - 1,)),
            pltpu.SemaphoreType.DMA((2,)),
            pltpu.SemaphoreType.DMA((2,)),
        ],
        compiler_params=pltpu.CompilerParams(
            collective_id=0,
            vmem_limit_bytes=64 * 1024 * 1024,
        ),
    )(x, w_mat, scale_x, scale_w)


# device time: 42128 ns/iter; 1.5883x vs baseline; 1.0406x over previous
import jax
import jax.numpy as jnp
from jax import lax
from jax.experimental import pallas as pl
from jax.experimental.pallas import tpu as pltpu

N_DEV = 4
NC = 2


def kernel(x, w_mat, scale_x, scale_w):
    m_total, k_shard = x.shape
    k_total, n = w_mat.shape
    m_per = m_total // N_DEV
    half = m_per // NC
    assert k_total == N_DEV * k_shard

    def body(x_hbm, w_hbm, sx_ref, sw_ref, out_ref,
             send_buf, recv_buf, own_buf, xtmp, wtmp,
             send_sems, recv_sems, xsems, wsems):
        my_i = lax.axis_index("i")
        scale = sx_ref[0] * sw_ref[0]

        barrier_sem = pltpu.get_barrier_semaphore()
        for h in range(1, N_DEV):
            peer = lax.rem(my_i + h, N_DEV)
            pl.semaphore_signal(
                barrier_sem, inc=1,
                device_id=(peer,), device_id_type=pl.DeviceIdType.MESH)

        chunks = [(h, c) for h in (1, 2, 3, 0) for c in range(NC)]

        def xchunk(i, h, c):
            src_dev = lax.rem(my_i + h, N_DEV)
            return pltpu.make_async_copy(
                x_hbm.at[pl.ds(src_dev * m_per + c * half, half), :],
                xtmp.at[i % 2], xsems.at[i % 2])

        xdma = [xchunk(i, h, c) for i, (h, c) in enumerate(chunks)]

        def wblk(h, slot):
            src = lax.rem(my_i - h + N_DEV, N_DEV)
            return pltpu.make_async_copy(
                w_hbm.at[pl.ds(src * k_shard, k_shard), :],
                wtmp.at[slot], wsems.at[slot])

        wdma = [wblk(0, 0), wblk(1, 1), wblk(2, 0), wblk(3, 1)]

        xdma[0].start()
        xdma[1].start()

        pl.semaphore_wait(barrier_sem, N_DEV - 1)

        rdmas = {}
        for i, (h, c) in enumerate(chunks):
            xdma[i].wait()
            val8 = xtmp[i % 2].astype(jnp.float8_e5m2)
            if h == 0:
                own_buf[c] = val8
            else:
                send_buf[h - 1, c] = val8
                rdma = pltpu.make_async_remote_copy(
                    src_ref=send_buf.at[h - 1, c],
                    dst_ref=recv_buf.at[h - 1, c],
                    send_sem=send_sems.at[h - 1, c],
                    recv_sem=recv_sems.at[h - 1, c],
                    device_id=(lax.rem(my_i + h, N_DEV),),
                    device_id_type=pl.DeviceIdType.MESH,
                )
                rdma.start()
                rdmas[(h, c)] = rdma
            if i + 2 < len(chunks):
                xdma[i + 2].start()
            if i == 1:
                wdma[0].start()
        wdma[1].start()

        wdma[0].wait()
        b = wtmp[0].astype(jnp.float8_e5m2)
        wdma[2].start()
        for c in range(NC):
            out_ref[c * half:(c + 1) * half, :] = jnp.dot(
                own_buf[c], b, preferred_element_type=jnp.float32)

        for h in range(1, N_DEV):
            wdma[h].wait()
            b = wtmp[h % 2].astype(jnp.float8_e5m2)
            if h == 1:
                wdma[3].start()
            for c in range(NC):
                rdmas[(h, c)].wait_recv()
                rows = pl.ds(c * half, half)
                out_ref[rows, :] = out_ref[rows, :] + jnp.dot(
                    recv_buf[h - 1, c], b,
                    preferred_element_type=jnp.float32)

        out_ref[:, :] = out_ref[:, :] * scale

        for rdma in rdmas.values():
            rdma.wait_send()

    return pl.pallas_call(
        body,
        out_shape=jax.ShapeDtypeStruct((m_per, n), jnp.float32),
        in_specs=[
            pl.BlockSpec(memory_space=pl.ANY),
            pl.BlockSpec(memory_space=pl.ANY),
            pl.BlockSpec(memory_space=pltpu.SMEM),
            pl.BlockSpec(memory_space=pltpu.SMEM),
        ],
        out_specs=pl.BlockSpec(memory_space=pltpu.VMEM),
        scratch_shapes=[
            pltpu.VMEM((N_DEV - 1, NC, half, k_shard), jnp.float8_e5m2),
            pltpu.VMEM((N_DEV - 1, NC, half, k_shard), jnp.float8_e5m2),
            pltpu.VMEM((NC, half, k_shard), jnp.float8_e5m2),
            pltpu.VMEM((2, half, k_shard), jnp.float32),
            pltpu.VMEM((2, k_shard, n), jnp.float32),
            pltpu.SemaphoreType.DMA((N_DEV - 1, NC)),
            pltpu.SemaphoreType.DMA((N_DEV - 1, NC)),
            pltpu.SemaphoreType.DMA((2,)),
            pltpu.SemaphoreType.DMA((2,)),
        ],
        compiler_params=pltpu.CompilerParams(
            collective_id=0,
            vmem_limit_bytes=64 * 1024 * 1024,
        ),
    )(x, w_mat, scale_x, scale_w)
